# baseline (speedup 1.0000x reference)
import jax
import jax.numpy as jnp
from jax import lax
from jax.experimental import pallas as pl
from jax.experimental.pallas import tpu as pltpu
from jax.experimental.pallas import tpu_sc as plsc

_B = 16384
_D = 32
_NT = 4
_NC = 2
_NS = 16
_NW = _NC * _NS
_BPW = _B // _NW
_L = 16
_NCHUNK = _BPW // _L


def _body(idx_hbm, t0, t1, t2, t3, out_hbm,
          idx_v, bufs, gsems, ssem):
    tables = (t0, t1, t2, t3)
    wid = lax.axis_index("s") * _NC + lax.axis_index("c")
    base = wid * _BPW
    pltpu.sync_copy(idx_hbm.at[pl.ds(base, _BPW)], idx_v)

    def chunk_gathers(i, _):
        idxvec = idx_v[pl.ds(i * _L, _L)]
        for t in range(_NT):
            pltpu.async_copy(
                tables[t].at[idxvec],
                bufs.at[t, pl.ds(i * _L, _L), :],
                gsems.at[t],
            )
        return ()

    lax.fori_loop(0, _NCHUNK, chunk_gathers, (), unroll=False)

    def drain_gathers(t, i):
        idxvec = idx_v[pl.ds(i * _L, _L)]
        pltpu.make_async_copy(
            tables[t].at[idxvec],
            bufs.at[t, pl.ds(i * _L, _L), :],
            gsems.at[t],
        ).wait()

    for t in range(_NT):
        lax.fori_loop(
            0, _NCHUNK, lambda i, _, t=t: (drain_gathers(t, i), ())[1], (),
            unroll=False,
        )
        pltpu.async_copy(
            bufs.at[t],
            out_hbm.at[pl.ds(base, _BPW), pl.ds(t * _D, _D)],
            ssem,
        )
    for t in range(_NT):
        pltpu.make_async_copy(
            bufs.at[t],
            out_hbm.at[pl.ds(base, _BPW), pl.ds(t * _D, _D)],
            ssem,
        ).wait()


def kernel(indexes, table0, table1, table2, table3):
    idx = indexes.astype(jnp.int32)
    f = pl.kernel(
        _body,
        out_type=jax.ShapeDtypeStruct((_B, _NT * _D), jnp.float32),
        mesh=plsc.VectorSubcoreMesh(core_axis_name="c", subcore_axis_name="s"),
        compiler_params=pltpu.CompilerParams(use_tc_tiling_on_sc=False),
        scratch_types=[
            pltpu.VMEM((_BPW,), jnp.int32),
            pltpu.VMEM((_NT, _BPW, _D), jnp.float32),
            pltpu.SemaphoreType.DMA((_NT,)),
            pltpu.SemaphoreType.DMA,
        ],
    )
    return f(idx, table0, table1, table2, table3)
